# baseline (speedup 1.0000x reference)
import functools

import jax
import jax.numpy as jnp
from jax import lax
from jax.experimental import pallas as pl
from jax.experimental.pallas import tpu as pltpu
from jax.experimental.pallas import tpu_sc as plsc

N = 10000
F = 128
TM = 200

_NC, _NS = 2, 16
_NW = _NC * _NS
_CHUNK = 128
_NCHUNK = 3
_PER_W = _CHUNK * _NCHUNK
_BPAD = _NW * _PER_W


def _sc_gather(table, idx3):
    fw = table.shape[1]
    mesh = plsc.VectorSubcoreMesh(core_axis_name="c", subcore_axis_name="s")

    @functools.partial(
        pl.kernel,
        mesh=mesh,
        out_type=jax.ShapeDtypeStruct((_BPAD, fw), jnp.float32),
        scratch_types=[
            pltpu.VMEM((_NCHUNK, _CHUNK), jnp.int32),
            pltpu.VMEM((_PER_W, fw), jnp.float32),
            pltpu.SemaphoreType.DMA,
        ],
    )
    def k(table_hbm, idx_hbm, out_hbm, idx_v, rows_v, sem):
        wid = lax.axis_index("s") * _NC + lax.axis_index("c")
        pltpu.sync_copy(idx_hbm.at[wid], idx_v)
        copies = [
            pltpu.async_copy(
                table_hbm.at[idx_v.at[c]],
                rows_v.at[pl.ds(c * _CHUNK, _CHUNK)],
                sem,
            )
            for c in range(_NCHUNK)
        ]
        for cp in copies:
            cp.wait()
        pltpu.sync_copy(rows_v, out_hbm.at[pl.ds(wid * _PER_W, _PER_W)])

    return k(table, idx3)


def _xw_body(x_ref, w_ref, xwf_ref):
    xwf_ref[...] = jnp.dot(x_ref[...], w_ref[...],
                           preferred_element_type=jnp.float32)


def _matmul_xw(X, W):
    return pl.pallas_call(
        _xw_body,
        grid=(5,),
        in_specs=[
            pl.BlockSpec((2000, F), lambda i: (i, 0)),
            pl.BlockSpec((F, F), lambda i: (0, 0)),
        ],
        out_specs=pl.BlockSpec((2000, F), lambda i: (i, 0)),
        out_shape=jax.ShapeDtypeStruct((N, F), jnp.float32),
    )(X, W)


def _main_body(ap_ref, a_ref, xw_ref, xwp_ref, h_ref, hn_ref, cs_ref):
    a = a_ref[...]
    c1 = jnp.dot(a, xw_ref[...], preferred_element_type=jnp.float32)
    c2 = jnp.dot(a, xwp_ref[...], preferred_element_type=jnp.float32)
    al = ap_ref[0]
    h1 = jnp.where(c1 >= 0, c1, al * c1)
    h2 = jnp.where(c2 >= 0, c2, al * c2)
    h_ref[...] = h1.astype(jnp.bfloat16)
    hn_ref[...] = h2.astype(jnp.bfloat16)
    cs_ref[...] = jnp.sum(h1, axis=0)[None, None, :]


def _main(A, XW, XWp, a_prelu):
    return pl.pallas_call(
        _main_body,
        grid=(N // TM,),
        in_specs=[
            pl.BlockSpec(memory_space=pltpu.SMEM),
            pl.BlockSpec((TM, N), lambda i: (i, 0)),
            pl.BlockSpec((N, F), lambda i: (0, 0)),
            pl.BlockSpec((N, F), lambda i: (0, 0)),
        ],
        out_specs=[
            pl.BlockSpec((TM, F), lambda i: (i, 0)),
            pl.BlockSpec((TM, F), lambda i: (i, 0)),
            pl.BlockSpec((1, 1, F), lambda i: (i, 0, 0)),
        ],
        out_shape=[
            jax.ShapeDtypeStruct((N, F), jnp.bfloat16),
            jax.ShapeDtypeStruct((N, F), jnp.bfloat16),
            jax.ShapeDtypeStruct((N // TM, 1, F), jnp.float32),
        ],
        compiler_params=pltpu.CompilerParams(
            dimension_semantics=("parallel",)),
    )(a_prelu.reshape(1), A, XW, XWp)


def _readout_body(h_ref, hn_ref, cs_ref, wfc_ref, o_ref):
    cs = jnp.sum(cs_ref[...], axis=(0, 1))[None, :]
    s = jax.nn.sigmoid(cs * (1.0 / N))
    x = jnp.sum(wfc_ref[...] * s, axis=1)
    o_ref[pl.ds(0, N)] = jnp.sum(h_ref[...] * x[None, :], axis=1)
    o_ref[pl.ds(N, N)] = jnp.sum(hn_ref[...] * x[None, :], axis=1)


def _readout(H, Hn, cs, Wfc):
    return pl.pallas_call(
        _readout_body,
        grid=(1,),
        in_specs=[
            pl.BlockSpec((N, F), lambda i: (0, 0)),
            pl.BlockSpec((N, F), lambda i: (0, 0)),
            pl.BlockSpec((N // TM, 1, F), lambda i: (0, 0, 0)),
            pl.BlockSpec((F, F), lambda i: (0, 0)),
        ],
        out_specs=pl.BlockSpec((2 * N,), lambda i: (0,)),
        out_shape=jax.ShapeDtypeStruct((2 * N,), jnp.float32),
    )(H, Hn, cs, Wfc)


def kernel(X, A, W, a_prelu, Wfc, perm):
    XWf = _matmul_xw(X, W)
    idx3 = jnp.concatenate(
        [perm.astype(jnp.int32),
         jnp.arange(_BPAD - N, dtype=jnp.int32)]).reshape(_NW, _NCHUNK, _CHUNK)
    XWp = _sc_gather(XWf, idx3)
    H, Hn, cs = _main(A, XWf, XWp, a_prelu)
    out = _readout(H, Hn, cs, Wfc)
    labels = jnp.concatenate([
        jnp.ones((N,), dtype=jnp.float32),
        jnp.zeros((N,), dtype=jnp.float32),
    ])
    return (out, labels, jnp.array(0.0, dtype=jnp.float32))


# speedup vs baseline: 1.1156x; 1.1156x over previous
import functools

import jax
import jax.numpy as jnp
from jax import lax
from jax.experimental import pallas as pl
from jax.experimental.pallas import tpu as pltpu
from jax.experimental.pallas import tpu_sc as plsc

N = 10000
F = 128
TM = 400

_NC, _NS = 2, 16
_NW = _NC * _NS
_CHUNK = 128
_NCHUNK = 3
_PER_W = _CHUNK * _NCHUNK
_BPAD = _NW * _PER_W


def _sc_gather(table, idx3):
    fw = table.shape[1]
    mesh = plsc.VectorSubcoreMesh(core_axis_name="c", subcore_axis_name="s")

    @functools.partial(
        pl.kernel,
        mesh=mesh,
        out_type=jax.ShapeDtypeStruct((_BPAD, fw), jnp.float32),
        scratch_types=[
            pltpu.VMEM((_NCHUNK, _CHUNK), jnp.int32),
            pltpu.VMEM((_PER_W, fw), jnp.float32),
            pltpu.SemaphoreType.DMA,
        ],
    )
    def k(table_hbm, idx_hbm, out_hbm, idx_v, rows_v, sem):
        wid = lax.axis_index("s") * _NC + lax.axis_index("c")
        pltpu.sync_copy(idx_hbm.at[wid], idx_v)
        copies = [
            pltpu.async_copy(
                table_hbm.at[idx_v.at[c]],
                rows_v.at[pl.ds(c * _CHUNK, _CHUNK)],
                sem,
            )
            for c in range(_NCHUNK)
        ]
        for cp in copies:
            cp.wait()
        pltpu.sync_copy(rows_v, out_hbm.at[pl.ds(wid * _PER_W, _PER_W)])

    return k(table, idx3)


def _xw_body(x_ref, w_ref, xwf_ref):
    xwf_ref[...] = jnp.dot(x_ref[...], w_ref[...],
                           preferred_element_type=jnp.float32)


def _matmul_xw(X, W):
    return pl.pallas_call(
        _xw_body,
        grid=(5,),
        in_specs=[
            pl.BlockSpec((2000, F), lambda i: (i, 0)),
            pl.BlockSpec((F, F), lambda i: (0, 0)),
        ],
        out_specs=pl.BlockSpec((2000, F), lambda i: (i, 0)),
        out_shape=jax.ShapeDtypeStruct((N, F), jnp.float32),
    )(X, W)


def _main_body(ap_ref, a_ref, xw_ref, xwp_ref, h_ref, hn_ref, cs_ref):
    a = a_ref[...]
    c1 = jnp.dot(a, xw_ref[...], preferred_element_type=jnp.float32)
    c2 = jnp.dot(a, xwp_ref[...], preferred_element_type=jnp.float32)
    al = ap_ref[0]
    h1 = jnp.where(c1 >= 0, c1, al * c1)
    h2 = jnp.where(c2 >= 0, c2, al * c2)
    h_ref[...] = h1.astype(jnp.bfloat16)
    hn_ref[...] = h2.astype(jnp.bfloat16)
    cs_ref[...] = jnp.sum(h1, axis=0)[None, None, :]


def _main(A, XW, XWp, a_prelu):
    return pl.pallas_call(
        _main_body,
        grid=(N // TM,),
        in_specs=[
            pl.BlockSpec(memory_space=pltpu.SMEM),
            pl.BlockSpec((TM, N), lambda i: (i, 0)),
            pl.BlockSpec((N, F), lambda i: (0, 0)),
            pl.BlockSpec((N, F), lambda i: (0, 0)),
        ],
        out_specs=[
            pl.BlockSpec((TM, F), lambda i: (i, 0)),
            pl.BlockSpec((TM, F), lambda i: (i, 0)),
            pl.BlockSpec((1, 1, F), lambda i: (i, 0, 0)),
        ],
        out_shape=[
            jax.ShapeDtypeStruct((N, F), jnp.bfloat16),
            jax.ShapeDtypeStruct((N, F), jnp.bfloat16),
            jax.ShapeDtypeStruct((N // TM, 1, F), jnp.float32),
        ],
        compiler_params=pltpu.CompilerParams(
            dimension_semantics=("parallel",)),
    )(a_prelu.reshape(1), A, XW, XWp)


def _readout_body(h_ref, hn_ref, cs_ref, wfc_ref, o_ref):
    cs = jnp.sum(cs_ref[...], axis=(0, 1))[None, :]
    s = jax.nn.sigmoid(cs * (1.0 / N))
    x = jnp.sum(wfc_ref[...] * s, axis=1)[None, :]
    xb = x.astype(jnp.bfloat16)
    o_ref[:, pl.ds(0, N)] = jax.lax.dot_general(
        xb, h_ref[...], (((1,), (1,)), ((), ())),
        preferred_element_type=jnp.float32)
    o_ref[:, pl.ds(N, N)] = jax.lax.dot_general(
        xb, hn_ref[...], (((1,), (1,)), ((), ())),
        preferred_element_type=jnp.float32)


def _readout(H, Hn, cs, Wfc):
    return pl.pallas_call(
        _readout_body,
        grid=(1,),
        in_specs=[
            pl.BlockSpec((N, F), lambda i: (0, 0)),
            pl.BlockSpec((N, F), lambda i: (0, 0)),
            pl.BlockSpec((N // TM, 1, F), lambda i: (0, 0, 0)),
            pl.BlockSpec((F, F), lambda i: (0, 0)),
        ],
        out_specs=pl.BlockSpec((1, 2 * N), lambda i: (0, 0)),
        out_shape=jax.ShapeDtypeStruct((1, 2 * N), jnp.float32),
    )(H, Hn, cs, Wfc)


def kernel(X, A, W, a_prelu, Wfc, perm):
    XWf = _matmul_xw(X, W)
    idx3 = jnp.concatenate(
        [perm.astype(jnp.int32),
         jnp.arange(_BPAD - N, dtype=jnp.int32)]).reshape(_NW, _NCHUNK, _CHUNK)
    XWp = _sc_gather(XWf, idx3)
    H, Hn, cs = _main(A, XWf, XWp, a_prelu)
    out = _readout(H, Hn, cs, Wfc).reshape(2 * N)
    labels = jnp.concatenate([
        jnp.ones((N,), dtype=jnp.float32),
        jnp.zeros((N,), dtype=jnp.float32),
    ])
    return (out, labels, jnp.array(0.0, dtype=jnp.float32))


# speedup vs baseline: 1.1183x; 1.0024x over previous
import functools

import jax
import jax.numpy as jnp
from jax import lax
from jax.experimental import pallas as pl
from jax.experimental.pallas import tpu as pltpu
from jax.experimental.pallas import tpu_sc as plsc

N = 10000
F = 128
TM = 400

_NC, _NS = 2, 16
_NW = _NC * _NS
_PER_W = 400
_NACT = N // _PER_W
_CHUNKS = (128, 128, 128, 16)


def _sc_gather(table, idx):
    fw = table.shape[1]
    mesh = plsc.VectorSubcoreMesh(core_axis_name="c", subcore_axis_name="s")

    @functools.partial(
        pl.kernel,
        mesh=mesh,
        out_type=jax.ShapeDtypeStruct((N, fw), jnp.float32),
        scratch_types=[
            pltpu.VMEM((_PER_W,), jnp.int32),
            pltpu.VMEM((_PER_W, fw), jnp.float32),
            pltpu.SemaphoreType.DMA,
        ],
    )
    def k(table_hbm, idx_hbm, out_hbm, idx_v, rows_v, sem):
        wid = lax.axis_index("s") * _NC + lax.axis_index("c")

        @pl.when(wid < _NACT)
        def _():
            base = wid * _PER_W
            pltpu.sync_copy(idx_hbm.at[pl.ds(base, _PER_W)], idx_v)
            off = 0
            copies = []
            for c in _CHUNKS:
                copies.append(pltpu.async_copy(
                    table_hbm.at[idx_v.at[pl.ds(off, c)]],
                    rows_v.at[pl.ds(off, c)],
                    sem,
                ))
                off += c
            for cp in copies:
                cp.wait()
            pltpu.sync_copy(rows_v, out_hbm.at[pl.ds(base, _PER_W)])

    return k(table, idx)


def _xw_body(x_ref, w_ref, xwf_ref):
    xwf_ref[...] = jnp.dot(x_ref[...], w_ref[...],
                           preferred_element_type=jnp.float32)


def _matmul_xw(X, W):
    return pl.pallas_call(
        _xw_body,
        grid=(5,),
        in_specs=[
            pl.BlockSpec((2000, F), lambda i: (i, 0)),
            pl.BlockSpec((F, F), lambda i: (0, 0)),
        ],
        out_specs=pl.BlockSpec((2000, F), lambda i: (i, 0)),
        out_shape=jax.ShapeDtypeStruct((N, F), jnp.float32),
    )(X, W)


def _main_body(ap_ref, a_ref, xw_ref, xwp_ref, h_ref, hn_ref, cs_ref):
    a = a_ref[...]
    c1 = jnp.dot(a, xw_ref[...], preferred_element_type=jnp.float32)
    c2 = jnp.dot(a, xwp_ref[...], preferred_element_type=jnp.float32)
    al = ap_ref[0]
    h1 = jnp.where(c1 >= 0, c1, al * c1)
    h2 = jnp.where(c2 >= 0, c2, al * c2)
    h_ref[...] = h1.astype(jnp.bfloat16)
    hn_ref[...] = h2.astype(jnp.bfloat16)
    cs_ref[...] = jnp.sum(h1, axis=0)[None, None, :]


def _main(A, XW, XWp, a_prelu):
    return pl.pallas_call(
        _main_body,
        grid=(N // TM,),
        in_specs=[
            pl.BlockSpec(memory_space=pltpu.SMEM),
            pl.BlockSpec((TM, N), lambda i: (i, 0)),
            pl.BlockSpec((N, F), lambda i: (0, 0)),
            pl.BlockSpec((N, F), lambda i: (0, 0)),
        ],
        out_specs=[
            pl.BlockSpec((TM, F), lambda i: (i, 0)),
            pl.BlockSpec((TM, F), lambda i: (i, 0)),
            pl.BlockSpec((1, 1, F), lambda i: (i, 0, 0)),
        ],
        out_shape=[
            jax.ShapeDtypeStruct((N, F), jnp.bfloat16),
            jax.ShapeDtypeStruct((N, F), jnp.bfloat16),
            jax.ShapeDtypeStruct((N // TM, 1, F), jnp.float32),
        ],
        compiler_params=pltpu.CompilerParams(
            dimension_semantics=("parallel",)),
    )(a_prelu.reshape(1), A, XW, XWp)


def _readout_body(h_ref, hn_ref, cs_ref, wfc_ref, o_ref, lab_ref):
    lab_ref[...] = jnp.where(
        jax.lax.broadcasted_iota(jnp.int32, (1, 2 * N), 1) < N, 1.0, 0.0
    ).astype(jnp.float32)
    cs = jnp.sum(cs_ref[...], axis=(0, 1))[None, :]
    s = jax.nn.sigmoid(cs * (1.0 / N))
    x = jnp.sum(wfc_ref[...] * s, axis=1)[None, :]
    xb = x.astype(jnp.bfloat16)
    o_ref[:, pl.ds(0, N)] = jax.lax.dot_general(
        xb, h_ref[...], (((1,), (1,)), ((), ())),
        preferred_element_type=jnp.float32)
    o_ref[:, pl.ds(N, N)] = jax.lax.dot_general(
        xb, hn_ref[...], (((1,), (1,)), ((), ())),
        preferred_element_type=jnp.float32)


def _readout(H, Hn, cs, Wfc):
    return pl.pallas_call(
        _readout_body,
        grid=(1,),
        in_specs=[
            pl.BlockSpec((N, F), lambda i: (0, 0)),
            pl.BlockSpec((N, F), lambda i: (0, 0)),
            pl.BlockSpec((N // TM, 1, F), lambda i: (0, 0, 0)),
            pl.BlockSpec((F, F), lambda i: (0, 0)),
        ],
        out_specs=[
            pl.BlockSpec((1, 2 * N), lambda i: (0, 0)),
            pl.BlockSpec((1, 2 * N), lambda i: (0, 0)),
        ],
        out_shape=[
            jax.ShapeDtypeStruct((1, 2 * N), jnp.float32),
            jax.ShapeDtypeStruct((1, 2 * N), jnp.float32),
        ],
    )(H, Hn, cs, Wfc)


def kernel(X, A, W, a_prelu, Wfc, perm):
    XWf = _matmul_xw(X, W)
    XWp = _sc_gather(XWf, perm)
    H, Hn, cs = _main(A, XWf, XWp, a_prelu)
    out, labels = _readout(H, Hn, cs, Wfc)
    return (out.reshape(2 * N), labels.reshape(2 * N),
            jnp.array(0.0, dtype=jnp.float32))
